# baseline (speedup 1.0000x reference)
import functools

import jax
import jax.numpy as jnp
import numpy as np
from jax import lax
from jax.experimental import pallas as pl
from jax.experimental.pallas import tpu as pltpu
from jax.experimental.pallas import tpu_sc as plsc

BS, C, H, W = 4, 384, 32, 32
HW = H * W
P, G = 9, 8
GC = C // G
NQ = BS * HW
NPC = P * 4
IDXW = 16

_PERM = (np.arange(C) % G) * GC + np.arange(C) // G

_k = np.arange(NPC * 16)
_pc = _k // 16
_p = _pc // 4
_c = _pc % 4
_l = _k % 16
_D1 = np.zeros((P * G, NPC * 16), np.float32)
_D1[_p * G + (_l % G), _k] = 1.0
_D2 = np.zeros((NPC, NPC * 16), np.float32)
_D2[_c * P + _p, _k] = 1.0
_ES = np.zeros((P * G, G), np.float32)
_ES[np.arange(P * G), np.arange(P * G) % G] = 1.0
_EB = np.zeros((G, P * G), np.float32)
_EB[np.arange(P * G) % G, np.arange(P * G)] = 1.0


def _prologue_body(f1_ref, f2_ref, anc_ref, g1_ref, b1_ref, g2_ref, b2_ref,
                   wv_ref, bv_ref, wwt_ref, bwt_ref,
                   wkx_ref, bkx_ref, wky_ref, bky_ref,
                   d1_ref, d2_ref, es_ref, eb_ref,
                   quad_out, cw_out, idx_out, kpx_out, kpy_out):
    b = pl.program_id(0)
    f32 = jnp.float32

    def ln(x, g, bb):
        m = jnp.mean(x, axis=-1, keepdims=True)
        v = jnp.mean((x - m) * (x - m), axis=-1, keepdims=True)
        return (x - m) * lax.rsqrt(v + 1e-5) * g + bb

    l1 = ln(f1_ref[0], g1_ref[...], b1_ref[...])
    l2 = ln(f2_ref[0], g2_ref[...], b2_ref[...])

    val = jnp.dot(l2, wv_ref[...], preferred_element_type=f32) + bv_ref[...]
    v1 = jnp.concatenate([val[1:], val[-1:]], axis=0)
    v32 = jnp.concatenate([val[32:], val[-32:]], axis=0)
    v33 = jnp.concatenate([val[33:], val[-33:]], axis=0)
    quad_out[0] = jnp.concatenate([val, v1, v32, v33], axis=1)

    wl = jnp.dot(l1, wwt_ref[...], preferred_element_type=f32) + bwt_ref[...]
    m = jnp.max(wl, axis=-1, keepdims=True)
    e = jnp.exp(wl - m)
    s8 = jnp.dot(e, es_ref[...], preferred_element_type=f32)
    den = jnp.dot(s8, eb_ref[...], preferred_element_type=f32)
    w72 = e / den

    offx = jnp.dot(l1, wkx_ref[...], preferred_element_type=f32) + bkx_ref[...]
    offy = jnp.dot(l1, wky_ref[...], preferred_element_type=f32) + bky_ref[...]
    anc = anc_ref[0]
    kx = anc[:, 0:1] + offx
    ky = anc[:, 1:2] + offy
    kpx_out[0] = kx
    kpy_out[0] = ky

    x = kx * W - 0.5
    y = ky * H - 0.5
    x0 = jnp.floor(x)
    y0 = jnp.floor(y)
    wx1 = x - x0
    wx0 = 1.0 - wx1
    wy1 = y - y0
    wy0 = 1.0 - wy1

    def slot_weights(c0, w0, w1, n):
        cb = jnp.clip(c0, 0, n - 1)
        c1 = c0 + 1.0
        v0 = ((c0 >= 0) & (c0 < n)).astype(jnp.float32)
        v1 = ((c1 >= 0) & (c1 < n)).astype(jnp.float32)
        ws0 = w0 * v0 * (c0 == cb) + w1 * v1 * (c1 == cb)
        ws1 = w0 * v0 * (c0 == cb + 1.0) + w1 * v1 * (c1 == cb + 1.0)
        return cb.astype(jnp.int32), ws0, ws1

    xb, wxs0, wxs1 = slot_weights(x0, wx0, wx1, W)
    yb, wys0, wys1 = slot_weights(y0, wy0, wy1, H)

    idxq = yb * W + xb + b * HW
    zpad = jnp.zeros((HW, IDXW - P), jnp.int32)
    idx_out[0] = jnp.concatenate([idxq, zpad], axis=1)

    bw36 = jnp.concatenate([wxs0 * wys0, wxs1 * wys0,
                            wxs0 * wys1, wxs1 * wys1], axis=1)

    cw_out[0] = (jnp.dot(w72, d1_ref[...], preferred_element_type=f32)
                 * jnp.dot(bw36, d2_ref[...], preferred_element_type=f32))


def _epilogue_body(agg_ref, w_ref, b_ref, out_ref):
    out_ref[...] = (jnp.dot(agg_ref[...], w_ref[...],
                            preferred_element_type=jnp.float32) + b_ref[...])


def _sc_agg_body(quad_hbm, idx_hbm, cw_hbm, out_hbm,
                 idx_all, cw_all, rows0, rows1, orow0, orow1,
                 sem0, sem1, osem0, osem1):
    nc = 2
    qper = NQ // 32
    wid = lax.axis_index("s") * nc + lax.axis_index("c")
    base = wid * qper
    pltpu.sync_copy(idx_hbm.at[pl.ds(base, qper)], idx_all)
    pltpu.sync_copy(cw_hbm.at[pl.ds(base, qper)], cw_all)

    rows_b = (rows0, rows1)
    orow_b = (orow0, orow1)
    sem_b = (sem0, sem1)
    osem_b = (osem0, osem1)

    def issue(i, b):
        pltpu.async_copy(quad_hbm.at[idx_all.at[i]], rows_b[b], sem_b[b])

    def drain(i, b):
        pltpu.make_async_copy(quad_hbm.at[idx_all.at[i]], rows_b[b],
                              sem_b[b]).wait()

    issue(0, 0)
    issue(1, 1)

    def outer(ii, carry):
        for b in range(2):
            i = ii * 2 + b
            drain(i, b)

            orow, osem = orow_b[b], osem_b[b]

            @pl.when(ii > 0)
            def _():
                pltpu.make_async_copy(orow, out_hbm.at[base + i - 2], osem).wait()

            rows = rows_b[b]
            wvec0 = cw_all[i, 0, :]
            for j in range(24):
                orow[pl.ds(j * 16, 16)] = rows[0, pl.ds(j * 16, 16)] * wvec0
            for c in range(1, 4):
                wvec = cw_all[i, c, :]
                for j in range(24):
                    plsc.addupdate(orow.at[pl.ds(j * 16, 16)],
                                   rows[0, pl.ds(c * C + j * 16, 16)] * wvec)

            def p_body(p, carry2):
                for c in range(4):
                    wvec = cw_all[i, p * 4 + c, :]
                    for j in range(24):
                        plsc.addupdate(orow.at[pl.ds(j * 16, 16)],
                                       rows[p, pl.ds(c * C + j * 16, 16)] * wvec)
                return carry2

            lax.fori_loop(1, P, p_body, 0)

            nxt = jnp.bitwise_and(i + 2, qper - 1)
            issue(nxt, b)
            pltpu.async_copy(orow, out_hbm.at[base + i], osem)
        return carry

    lax.fori_loop(0, qper // 2, outer, 0)
    for b in range(2):
        pltpu.make_async_copy(orow_b[b], out_hbm.at[base + qper - 2 + b],
                              osem_b[b]).wait()
        drain(b, b)


def kernel(feats1, feats2, anchor_points, ln1_g, ln1_b, ln2_g, ln2_b,
           W_val, b_val, W_wt, b_wt, W_kp, b_kp, W_out, b_out):
    f32 = jnp.float32
    perm = jnp.asarray(_PERM)

    f1 = feats1.transpose(0, 2, 3, 1).reshape(BS, HW, C)
    f2 = feats2.transpose(0, 2, 3, 1).reshape(BS, HW, C)

    wv = W_val[perm].T
    bv = b_val[perm].reshape(1, C)
    wwt = W_wt.T
    bwt = b_wt.reshape(1, P * G)
    wkx = W_kp[:, 0::2]
    wky = W_kp[:, 1::2]
    bkx = b_kp[0::2].reshape(1, P)
    bky = b_kp[1::2].reshape(1, P)
    wo = W_out[:, perm].T
    bo = b_out.reshape(1, C)

    full = lambda shape: pl.BlockSpec(shape, lambda b: tuple(0 for _ in shape))
    per_b = lambda shape: pl.BlockSpec((1,) + shape, lambda b: (b, 0, 0))

    quad, cw, idx, kpx, kpy = pl.pallas_call(
        _prologue_body,
        grid=(BS,),
        in_specs=[
            per_b((HW, C)), per_b((HW, C)), per_b((HW, 2)),
            full((1, C)), full((1, C)), full((1, C)), full((1, C)),
            full((C, C)), full((1, C)),
            full((C, P * G)), full((1, P * G)),
            full((C, P)), full((1, P)), full((C, P)), full((1, P)),
            full((P * G, NPC * 16)), full((NPC, NPC * 16)),
            full((P * G, G)), full((G, P * G)),
        ],
        out_specs=[
            per_b((HW, 4 * C)), per_b((HW, NPC * 16)), per_b((HW, IDXW)),
            per_b((HW, P)), per_b((HW, P)),
        ],
        out_shape=[
            jax.ShapeDtypeStruct((BS, HW, 4 * C), f32),
            jax.ShapeDtypeStruct((BS, HW, NPC * 16), f32),
            jax.ShapeDtypeStruct((BS, HW, IDXW), jnp.int32),
            jax.ShapeDtypeStruct((BS, HW, P), f32),
            jax.ShapeDtypeStruct((BS, HW, P), f32),
        ],
    )(f1, f2, anchor_points,
      ln1_g.reshape(1, C), ln1_b.reshape(1, C),
      ln2_g.reshape(1, C), ln2_b.reshape(1, C),
      wv, bv, wwt, bwt, wkx, bkx, wky, bky,
      jnp.asarray(_D1), jnp.asarray(_D2), jnp.asarray(_ES), jnp.asarray(_EB))

    mesh = plsc.VectorSubcoreMesh(core_axis_name="c", subcore_axis_name="s",
                                  num_cores=2, num_subcores=16)
    qper = NQ // 32
    agg = pl.kernel(
        _sc_agg_body,
        out_type=jax.ShapeDtypeStruct((NQ, C), f32),
        mesh=mesh,
        compiler_params=pltpu.CompilerParams(use_tc_tiling_on_sc=False),
        scratch_types=[
            pltpu.VMEM((qper, IDXW), jnp.int32),
            pltpu.VMEM((qper, NPC, 16), f32),
            pltpu.VMEM((IDXW, 4 * C), f32),
            pltpu.VMEM((IDXW, 4 * C), f32),
            pltpu.VMEM((C,), f32),
            pltpu.VMEM((C,), f32),
            pltpu.SemaphoreType.DMA,
            pltpu.SemaphoreType.DMA,
            pltpu.SemaphoreType.DMA,
            pltpu.SemaphoreType.DMA,
        ],
    )(quad.reshape(NQ, 4 * C), idx.reshape(NQ, IDXW),
      cw.reshape(NQ, NPC, 16))

    out2d = pl.pallas_call(
        _epilogue_body,
        in_specs=[pl.BlockSpec((NQ, C), lambda: (0, 0)),
                  pl.BlockSpec((C, C), lambda: (0, 0)),
                  pl.BlockSpec((1, C), lambda: (0, 0))],
        out_specs=pl.BlockSpec((NQ, C), lambda: (0, 0)),
        out_shape=jax.ShapeDtypeStruct((NQ, C), f32),
    )(agg, wo, bo)

    out = out2d.reshape(BS, H, W, C).transpose(0, 3, 1, 2)
    kp = jnp.stack([kpx, kpy], axis=-1).reshape(BS, H, W, P, 2)
    return out, kp


# speedup vs baseline: 2.7769x; 2.7769x over previous
import functools

import jax
import jax.numpy as jnp
import numpy as np
from jax import lax
from jax.experimental import pallas as pl
from jax.experimental.pallas import tpu as pltpu
from jax.experimental.pallas import tpu_sc as plsc

BS, C, H, W = 4, 384, 32, 32
HW = H * W
P, G = 9, 8
GC = C // G
NQ = BS * HW
NPC = P * 4
IDXW = 40

_m = np.arange(C)
_jj = _m // 32
_kk = (_m % 32) // 2
_hh = _m % 2
_SIGMA = (_kk % G) * GC + (_jj * 4 + (_kk // G) * 2 + _hh)
_n = np.arange(C)
_AGG = _SIGMA[32 * (_n // 32) + 2 * (_n % 16) + ((_n % 32) // 16)]

_k = np.arange(NPC * 16)
_pc = _k // 16
_p = _pc % P
_c = _pc // P
_l = _k % 16
_D1 = np.zeros((P * G, NPC * 16), np.float32)
_D1[_p * G + (_l % G), _k] = 1.0
_D2 = np.zeros((NPC, NPC * 16), np.float32)
_D2[_c * P + _p, _k] = 1.0
_ES = np.zeros((P * G, G), np.float32)
_ES[np.arange(P * G), np.arange(P * G) % G] = 1.0
_EB = np.zeros((G, P * G), np.float32)
_EB[np.arange(P * G) % G, np.arange(P * G)] = 1.0


def _prologue_body(f1_ref, f2_ref, anc_ref, g1_ref, b1_ref, g2_ref, b2_ref,
                   wv_ref, bv_ref, wwt_ref, bwt_ref,
                   wkx_ref, bkx_ref, wky_ref, bky_ref,
                   d1_ref, d2_ref, es_ref, eb_ref,
                   val_out, cw_out, idx_out, kpx_out, kpy_out):
    b = pl.program_id(0)
    f32 = jnp.float32

    def ln(x, g, bb):
        m = jnp.mean(x, axis=-1, keepdims=True)
        v = jnp.mean((x - m) * (x - m), axis=-1, keepdims=True)
        return (x - m) * lax.rsqrt(v + 1e-5) * g + bb

    l1 = ln(f1_ref[0], g1_ref[...], b1_ref[...])
    l2 = ln(f2_ref[0], g2_ref[...], b2_ref[...])

    val = jnp.dot(l2, wv_ref[...], preferred_element_type=f32) + bv_ref[...]
    val_out[0] = val.astype(jnp.bfloat16)

    wl = jnp.dot(l1, wwt_ref[...], preferred_element_type=f32) + bwt_ref[...]
    m = jnp.max(wl, axis=-1, keepdims=True)
    e = jnp.exp(wl - m)
    s8 = jnp.dot(e, es_ref[...], preferred_element_type=f32)
    den = jnp.dot(s8, eb_ref[...], preferred_element_type=f32)
    w72 = e / den

    offx = jnp.dot(l1, wkx_ref[...], preferred_element_type=f32) + bkx_ref[...]
    offy = jnp.dot(l1, wky_ref[...], preferred_element_type=f32) + bky_ref[...]
    anc = anc_ref[0]
    kx = anc[:, 0:1] + offx
    ky = anc[:, 1:2] + offy
    kpx_out[0] = kx
    kpy_out[0] = ky

    x = kx * W - 0.5
    y = ky * H - 0.5
    x0 = jnp.floor(x)
    y0 = jnp.floor(y)
    wx1 = x - x0
    wx0 = 1.0 - wx1
    wy1 = y - y0
    wy0 = 1.0 - wy1

    def corner(xf, yf, wx, wy):
        valid = (xf >= 0) & (xf < W) & (yf >= 0) & (yf < H)
        xi = jnp.clip(xf, 0, W - 1).astype(jnp.int32)
        yi = jnp.clip(yf, 0, H - 1).astype(jnp.int32)
        idx = yi * W + xi + b * HW
        return idx, wx * wy * valid.astype(f32)

    i0, w0 = corner(x0, y0, wx0, wy0)
    i1, w1 = corner(x0 + 1.0, y0, wx1, wy0)
    i2, w2 = corner(x0, y0 + 1.0, wx0, wy1)
    i3, w3 = corner(x0 + 1.0, y0 + 1.0, wx1, wy1)

    zpad = jnp.zeros((HW, IDXW - NPC), jnp.int32)
    idx_out[0] = jnp.concatenate([i0, i1, i2, i3, zpad], axis=1)
    bw36 = jnp.concatenate([w0, w1, w2, w3], axis=1)

    cw_out[0] = (jnp.dot(w72, d1_ref[...], preferred_element_type=f32)
                 * jnp.dot(bw36, d2_ref[...], preferred_element_type=f32))


def _epilogue_body(agg_ref, w_ref, b_ref, out_ref):
    out_ref[...] = (jnp.dot(agg_ref[...], w_ref[...],
                            preferred_element_type=jnp.float32) + b_ref[...])


def _sc_agg_body(value_hbm, idx_hbm, cw_hbm, out_hbm,
                 idx0, idx1, cw0, cw1, rows0, rows1, orow0, orow1, vshared,
                 msem0, msem1, sem0, sem1, osem0, osem1):
    nc = 2
    qper = NQ // 32
    sid = lax.axis_index("s")
    wid = sid * nc + lax.axis_index("c")
    base = wid * qper
    part = NQ // 16
    pltpu.sync_copy(value_hbm.at[pl.ds(sid * part, part)],
                    vshared.at[pl.ds(sid * part, part)])
    plsc.subcore_barrier()

    idx_b = (idx0, idx1)
    cw_b = (cw0, cw1)
    rows_b = (rows0, rows1)
    orow_b = (orow0, orow1)
    msem_b = (msem0, msem1)
    sem_b = (sem0, sem1)
    osem_b = (osem0, osem1)

    def issue_meta(i, b):
        pltpu.async_copy(idx_hbm.at[base + i], idx_b[b], msem_b[b])
        pltpu.async_copy(cw_hbm.at[base + i], cw_b[b], msem_b[b])

    def wait_meta(i, b):
        pltpu.make_async_copy(idx_hbm.at[base + i], idx_b[b], msem_b[b]).wait()
        pltpu.make_async_copy(cw_hbm.at[base + i], cw_b[b], msem_b[b]).wait()

    def issue_gather(b):
        pltpu.async_copy(vshared.at[idx_b[b]], rows_b[b], sem_b[b])

    def wait_gather(b):
        pltpu.make_async_copy(vshared.at[idx_b[b]], rows_b[b], sem_b[b]).wait()

    issue_meta(0, 0)
    issue_meta(1, 1)
    wait_meta(0, 0)
    issue_gather(0)

    def outer(ii, carry):
        for b in range(2):
            i = ii * 2 + b
            b1 = 1 - b
            wait_gather(b)
            nxt1 = jnp.bitwise_and(i + 1, qper - 1)
            wait_meta(nxt1, b1)
            issue_gather(b1)

            orow, osem = orow_b[b], osem_b[b]

            @pl.when(ii > 0)
            def _():
                pltpu.make_async_copy(orow, out_hbm.at[base + i - 2], osem).wait()

            rows, cw = rows_b[b], cw_b[b]
            wvec0 = cw[0, :]
            for j in range(12):
                ea, ob = plsc.unpack(rows[0, pl.ds(j * 32, 32)],
                                     format=plsc.PackFormat.INTERLEAVED)
                orow[pl.ds(j * 32, 16)] = ea * wvec0
                orow[pl.ds(j * 32 + 16, 16)] = ob * wvec0

            def pc_body(pc, c):
                wvec = cw[pc, :]
                for j in range(12):
                    ea, ob = plsc.unpack(rows[pc, pl.ds(j * 32, 32)],
                                         format=plsc.PackFormat.INTERLEAVED)
                    plsc.addupdate(orow.at[pl.ds(j * 32, 16)], ea * wvec)
                    plsc.addupdate(orow.at[pl.ds(j * 32 + 16, 16)], ob * wvec)
                return c

            lax.fori_loop(1, NPC, pc_body, 0)

            nxt2 = jnp.bitwise_and(i + 2, qper - 1)
            issue_meta(nxt2, b)
            pltpu.async_copy(orow, out_hbm.at[base + i], osem)
        return carry

    lax.fori_loop(0, qper // 2, outer, 0)
    for b in range(2):
        pltpu.make_async_copy(orow_b[b], out_hbm.at[base + qper - 2 + b],
                              osem_b[b]).wait()
    wait_gather(0)
    wait_meta(1, 1)


def kernel(feats1, feats2, anchor_points, ln1_g, ln1_b, ln2_g, ln2_b,
           W_val, b_val, W_wt, b_wt, W_kp, b_kp, W_out, b_out):
    f32 = jnp.float32
    sigma = jnp.asarray(_SIGMA)

    f1 = feats1.transpose(0, 2, 3, 1).reshape(BS, HW, C)
    f2 = feats2.transpose(0, 2, 3, 1).reshape(BS, HW, C)

    wv = W_val[sigma].T
    bv = b_val[sigma].reshape(1, C)
    wwt = W_wt.T
    bwt = b_wt.reshape(1, P * G)
    wkx = W_kp[:, 0::2]
    wky = W_kp[:, 1::2]
    bkx = b_kp[0::2].reshape(1, P)
    bky = b_kp[1::2].reshape(1, P)
    wo = W_out[:, jnp.asarray(_AGG)].T
    bo = b_out.reshape(1, C)

    full = lambda shape: pl.BlockSpec(shape, lambda b: tuple(0 for _ in shape))
    per_b = lambda shape: pl.BlockSpec((1,) + shape, lambda b: (b, 0, 0))

    value, cw, idx, kpx, kpy = pl.pallas_call(
        _prologue_body,
        grid=(BS,),
        in_specs=[
            per_b((HW, C)), per_b((HW, C)), per_b((HW, 2)),
            full((1, C)), full((1, C)), full((1, C)), full((1, C)),
            full((C, C)), full((1, C)),
            full((C, P * G)), full((1, P * G)),
            full((C, P)), full((1, P)), full((C, P)), full((1, P)),
            full((P * G, NPC * 16)), full((NPC, NPC * 16)),
            full((P * G, G)), full((G, P * G)),
        ],
        out_specs=[
            per_b((HW, C)), per_b((HW, NPC * 16)), per_b((HW, IDXW)),
            per_b((HW, P)), per_b((HW, P)),
        ],
        out_shape=[
            jax.ShapeDtypeStruct((BS, HW, C), jnp.bfloat16),
            jax.ShapeDtypeStruct((BS, HW, NPC * 16), f32),
            jax.ShapeDtypeStruct((BS, HW, IDXW), jnp.int32),
            jax.ShapeDtypeStruct((BS, HW, P), f32),
            jax.ShapeDtypeStruct((BS, HW, P), f32),
        ],
    )(f1, f2, anchor_points,
      ln1_g.reshape(1, C), ln1_b.reshape(1, C),
      ln2_g.reshape(1, C), ln2_b.reshape(1, C),
      wv, bv, wwt, bwt, wkx, bkx, wky, bky,
      jnp.asarray(_D1), jnp.asarray(_D2), jnp.asarray(_ES), jnp.asarray(_EB))

    mesh = plsc.VectorSubcoreMesh(core_axis_name="c", subcore_axis_name="s",
                                  num_cores=2, num_subcores=16)
    agg = pl.kernel(
        _sc_agg_body,
        out_type=jax.ShapeDtypeStruct((NQ, C), f32),
        mesh=mesh,
        compiler_params=pltpu.CompilerParams(use_tc_tiling_on_sc=False,
                                             needs_layout_passes=False),
        scratch_types=[
            pltpu.VMEM((IDXW,), jnp.int32),
            pltpu.VMEM((IDXW,), jnp.int32),
            pltpu.VMEM((NPC, 16), f32),
            pltpu.VMEM((NPC, 16), f32),
            pltpu.VMEM((IDXW, C), jnp.bfloat16),
            pltpu.VMEM((IDXW, C), jnp.bfloat16),
            pltpu.VMEM((C,), f32),
            pltpu.VMEM((C,), f32),
            pltpu.VMEM_SHARED((NQ, C), jnp.bfloat16),
            pltpu.SemaphoreType.DMA,
            pltpu.SemaphoreType.DMA,
            pltpu.SemaphoreType.DMA,
            pltpu.SemaphoreType.DMA,
            pltpu.SemaphoreType.DMA,
            pltpu.SemaphoreType.DMA,
        ],
    )(value.reshape(NQ, C), idx.reshape(NQ, IDXW), cw.reshape(NQ, NPC, 16))

    out2d = pl.pallas_call(
        _epilogue_body,
        in_specs=[pl.BlockSpec((NQ, C), lambda: (0, 0)),
                  pl.BlockSpec((C, C), lambda: (0, 0)),
                  pl.BlockSpec((1, C), lambda: (0, 0))],
        out_specs=pl.BlockSpec((NQ, C), lambda: (0, 0)),
        out_shape=jax.ShapeDtypeStruct((NQ, C), f32),
    )(agg, wo, bo)

    out = out2d.reshape(BS, H, W, C).transpose(0, 3, 1, 2)
    kp = jnp.stack([kpx, kpy], axis=-1).reshape(BS, H, W, P, 2)
    return out, kp
